# baseline (speedup 1.0000x reference)
import functools

import jax
import jax.numpy as jnp
from jax import lax
from jax.experimental import pallas as pl
from jax.experimental.pallas import tpu as pltpu
from jax.experimental.pallas import tpu_sc as plsc

_B, _L, _D = 4096, 200, 64
_DP = 128
_N = _B * _L
_NC, _NS = 2, 16
_NW = _NC * _NS
_PER_W = _B // _NW
_IDX_W = _PER_W * _L
_NBUF = 4
_ROWS_PER = 2
_CHUNK = _ROWS_PER * _L
_NCH = _PER_W // _ROWS_PER
_ROUNDS = _NCH // _NBUF

_mesh = plsc.VectorSubcoreMesh(core_axis_name="c", subcore_axis_name="s")


@functools.partial(
    pl.kernel,
    out_type=jax.ShapeDtypeStruct((_B, _L, _DP), jnp.float32),
    mesh=_mesh,
    scratch_types=(
        [pltpu.VMEM((_IDX_W,), jnp.int32)]
        + [pltpu.VMEM((_CHUNK, _D), jnp.float32) for _ in range(_NBUF)]
        + [pltpu.SemaphoreType.DMA for _ in range(2 * _NBUF)]
    ),
    compiler_params=pltpu.CompilerParams(use_tc_tiling_on_sc=False),
)
def _gather_kernel(x_hbm, table_hbm, out_hbm, idx_v, *rest):
    bufs = rest[:_NBUF]
    gsem = rest[_NBUF:2 * _NBUF]
    wsem = rest[2 * _NBUF:]

    wid = lax.axis_index("s") * _NC + lax.axis_index("c")
    base = pl.multiple_of(wid * _PER_W, _PER_W)
    idx_base = pl.multiple_of(wid * _IDX_W, _IDX_W)
    pltpu.sync_copy(x_hbm.at[pl.ds(idx_base, _IDX_W)], idx_v)

    def gather(i, b):
        off = pl.multiple_of(i * _CHUNK, 8)
        return pltpu.make_async_copy(
            table_hbm.at[idx_v.at[pl.ds(off, _CHUNK)]], bufs[b], gsem[b])

    def writes(i, b):
        row = base + i * _ROWS_PER
        return [
            pltpu.make_async_copy(
                bufs[b].at[pl.ds(j * _L, _L)],
                out_hbm.at[row + j, :, pl.ds(0, _D)], wsem[b])
            for j in range(_ROWS_PER)
        ]

    for b in range(_NBUF):
        gather(b, b).start()

    def round_body(r, carry):
        i0 = r * _NBUF
        for b in range(_NBUF):
            gather(i0 + b, b).wait()
            for w in writes(i0 + b, b):
                w.start()
        for b in range(_NBUF):
            for w in writes(i0 + b, b):
                w.wait()
            gather(i0 + _NBUF + b, b).start()
        return carry

    lax.fori_loop(0, _ROUNDS - 1, round_body, 0)

    i0 = (_ROUNDS - 1) * _NBUF
    for b in range(_NBUF):
        gather(i0 + b, b).wait()
        for w in writes(i0 + b, b):
            w.start()
    for b in range(_NBUF):
        for w in writes(i0 + b, b):
            w.wait()


def kernel(x, table):
    out = _gather_kernel(x.reshape(_N).astype(jnp.int32), table)
    return out[:, :, :_D]


# speedup vs baseline: 1.0073x; 1.0073x over previous
import functools

import jax
import jax.numpy as jnp
from jax import lax
from jax.experimental import pallas as pl
from jax.experimental.pallas import tpu as pltpu
from jax.experimental.pallas import tpu_sc as plsc

_B, _L, _D = 4096, 200, 64
_DP = 128
_N = _B * _L
_NC, _NS = 2, 16
_NW = _NC * _NS
_PER_W = _B // _NW
_IDX_W = _PER_W * _L
_NBUF = 4
_ROUNDS = _PER_W // _NBUF

_mesh = plsc.VectorSubcoreMesh(core_axis_name="c", subcore_axis_name="s")


@functools.partial(
    pl.kernel,
    out_type=jax.ShapeDtypeStruct((_B, _L, _DP), jnp.float32),
    mesh=_mesh,
    scratch_types=(
        [pltpu.VMEM((_IDX_W,), jnp.int32)]
        + [pltpu.VMEM((_L, _D), jnp.float32) for _ in range(_NBUF)]
        + [pltpu.SemaphoreType.DMA for _ in range(2 * _NBUF)]
    ),
    compiler_params=pltpu.CompilerParams(use_tc_tiling_on_sc=False),
)
def _gather_kernel(x_hbm, table_hbm, out_hbm, idx_v, *rest):
    bufs = rest[:_NBUF]
    gsem = rest[_NBUF:2 * _NBUF]
    wsem = rest[2 * _NBUF:]

    wid = lax.axis_index("s") * _NC + lax.axis_index("c")
    base = pl.multiple_of(wid * _PER_W, _PER_W)
    idx_base = pl.multiple_of(wid * _IDX_W, _IDX_W)
    pltpu.sync_copy(x_hbm.at[pl.ds(idx_base, _IDX_W)], idx_v)

    def gather(i, b):
        off = pl.multiple_of(i * _L, 8)
        return pltpu.make_async_copy(
            table_hbm.at[idx_v.at[pl.ds(off, _L)]], bufs[b], gsem[b])

    def write(i, b):
        return pltpu.make_async_copy(
            bufs[b], out_hbm.at[base + i, :, pl.ds(0, _D)], wsem[b])

    for b in range(_NBUF):
        gather(b, b).start()

    def round_body(r, carry):
        i0 = r * _NBUF
        for b in range(_NBUF):
            gather(i0 + b, b).wait()
            write(i0 + b, b).start()
        for b in range(_NBUF):
            write(i0 + b, b).wait()
            gather(i0 + _NBUF + b, b).start()
        return carry

    lax.fori_loop(0, _ROUNDS - 1, round_body, 0)

    i0 = (_ROUNDS - 1) * _NBUF
    for b in range(_NBUF):
        gather(i0 + b, b).wait()
        write(i0 + b, b).start()
    for b in range(_NBUF):
        write(i0 + b, b).wait()


def kernel(x, table):
    out = _gather_kernel(x.reshape(_N).astype(jnp.int32), table)
    return out[:, :, :_D]
